# baseline (speedup 1.0000x reference)
import math

import jax
import jax.numpy as jnp
import numpy as np
from jax import lax
from jax.experimental import pallas as pl
from jax.experimental.pallas import tpu as pltpu
from jax.experimental.pallas import tpu_sc as plsc

D_MODEL = 768
_SCALE = math.sqrt(float(D_MODEL))
_LANES = 16
_CHUNK = 16


def _pos_encoding(seq_len: int, d_model: int) -> np.ndarray:
    pos = np.arange(seq_len, dtype=np.float32)[:, None]
    i = np.arange(d_model, dtype=np.float32)[None, :]
    angle_rates = 1.0 / np.power(10000.0, (2.0 * np.floor(i / 2.0)) / d_model)
    angles = pos * angle_rates
    pe = np.zeros((seq_len, d_model), dtype=np.float32)
    pe[:, 0::2] = np.sin(angles[:, 0::2])
    pe[:, 1::2] = np.cos(angles[:, 1::2])
    return pe


def _make_sc_call(B: int, T: int, V: int, D: int):
    info = plsc.get_sparse_core_info()
    NC, NS = info.num_cores, info.num_subcores
    NW = NC * NS
    assert T % NW == 0
    t_per_w = T // NW
    assert t_per_w % _CHUNK == 0
    halves = t_per_w // _CHUNK
    n_chunks = B * halves

    mesh = plsc.VectorSubcoreMesh(core_axis_name="c", subcore_axis_name="s")

    @jax.jit
    def call(idx_w, table, pe):
        @pl.kernel(
            mesh=mesh,
            out_type=jax.ShapeDtypeStruct((B * T, D), jnp.float32),
            scratch_types=[
                pltpu.VMEM((B, t_per_w), jnp.int32),
                pltpu.VMEM((_CHUNK, D), jnp.float32),
                pltpu.VMEM((_CHUNK, D), jnp.float32),
                pltpu.VMEM((_CHUNK, D), jnp.float32),
                pltpu.VMEM((_CHUNK, D), jnp.float32),
                pltpu.VMEM((_CHUNK, D), jnp.float32),
                pltpu.VMEM_SHARED((NS, t_per_w, D), jnp.float32),
            ] + [pltpu.SemaphoreType.DMA] * 8,
        )
        def k(idx_hbm, table_hbm, pe_hbm, out_hbm,
              idx_v, g0, g1, o0, o1, o2, pe_sh,
              sg0, sg1, sf0, sf1, sf2, ss0, ss1, ss2):
            cid = lax.axis_index("c")
            sid = lax.axis_index("s")
            wid = sid * NC + cid
            t0 = wid * t_per_w
            pltpu.sync_copy(idx_hbm.at[wid], idx_v)

            gbuf, gsem = (g0, g1), (sg0, sg1)
            obuf, fsem = (o0, o1, o2), (sf0, sf1, sf2)
            ssem = (ss0, ss1, ss2)

            def loc(c):
                return divmod(c, halves)

            def gather_start(c):
                b, half = loc(c)
                idx = idx_v.at[b, pl.ds(half * _CHUNK, _CHUNK)]
                return pltpu.async_copy(table_hbm.at[idx], gbuf[c % 2],
                                        gsem[c % 2])

            def fill_start(c):
                _, half = loc(c)
                src = pe_sh.at[sid, pl.ds(half * _CHUNK, _CHUNK)]
                return pltpu.async_copy(src, obuf[c % 3], fsem[c % 3])

            def store_start(c):
                b, half = loc(c)
                dst = out_hbm.at[pl.ds(b * T + t0 + half * _CHUNK, _CHUNK)]
                return pltpu.async_copy(obuf[c % 3], dst, ssem[c % 3])

            h_g = [None] * n_chunks
            h_f = [None] * n_chunks
            h_s = [None] * n_chunks
            h_g[0] = gather_start(0)
            h_g[1] = gather_start(1)
            pltpu.sync_copy(pe_hbm.at[pl.ds(t0, t_per_w)], pe_sh.at[sid])
            h_f[0] = fill_start(0)
            h_f[1] = fill_start(1)

            for c in range(n_chunks):
                h_g[c].wait()
                h_f[c].wait()
                g, o = gbuf[c % 2], obuf[c % 3]

                def body(r2, _):
                    r = r2 * 2
                    for j in range(D // _LANES):
                        sl = pl.ds(j * _LANES, _LANES)
                        plsc.addupdate(o.at[r, sl], g[r, sl] * _SCALE)
                        plsc.addupdate(o.at[r + 1, sl], g[r + 1, sl] * _SCALE)
                    return _

                lax.fori_loop(0, _CHUNK // 2, body, None)
                h_s[c] = store_start(c)
                if c + 2 < n_chunks:
                    h_g[c + 2] = gather_start(c + 2)
                    if c >= 1:
                        h_s[c - 1].wait()
                    h_f[c + 2] = fill_start(c + 2)
            h_s[n_chunks - 3].wait()
            h_s[n_chunks - 2].wait()
            h_s[n_chunks - 1].wait()

        return k(idx_w, table, pe)

    return call


def kernel(token_ids, embedding_table):
    B, T = token_ids.shape
    V, D = embedding_table.shape
    info = plsc.get_sparse_core_info()
    NW = info.num_cores * info.num_subcores
    t_per_w = T // NW
    idx_w = token_ids.reshape(B, NW, t_per_w).transpose(1, 0, 2)
    pe = jnp.asarray(_pos_encoding(T, D))
    call = _make_sc_call(B, T, V, D)
    out = call(idx_w, embedding_table, pe)
    return out.reshape(B, T, D)


# speedup vs baseline: 1.6788x; 1.6788x over previous
import math

import jax
import jax.numpy as jnp
import numpy as np
from jax import lax
from jax.experimental import pallas as pl
from jax.experimental.pallas import tpu as pltpu
from jax.experimental.pallas import tpu_sc as plsc

D_MODEL = 768
_SCALE = math.sqrt(float(D_MODEL))
_LANES = 16
_CHUNK = 16


def _pos_encoding(seq_len: int, d_model: int) -> np.ndarray:
    pos = np.arange(seq_len, dtype=np.float32)[:, None]
    i = np.arange(d_model, dtype=np.float32)[None, :]
    angle_rates = 1.0 / np.power(10000.0, (2.0 * np.floor(i / 2.0)) / d_model)
    angles = pos * angle_rates
    pe = np.zeros((seq_len, d_model), dtype=np.float32)
    pe[:, 0::2] = np.sin(angles[:, 0::2])
    pe[:, 1::2] = np.cos(angles[:, 1::2])
    return pe


def _make_sc_call(B: int, T: int, V: int, D: int):
    info = plsc.get_sparse_core_info()
    NC, NS = info.num_cores, info.num_subcores
    NW = NC * NS
    assert T % NW == 0
    t_per_w = T // NW
    assert t_per_w % _CHUNK == 0
    halves = t_per_w // _CHUNK
    n_chunks = B * halves

    mesh = plsc.VectorSubcoreMesh(core_axis_name="c", subcore_axis_name="s")

    @jax.jit
    def call(idx_w, table, pe):
        @pl.kernel(
            mesh=mesh,
            out_type=jax.ShapeDtypeStruct((B * T, D), jnp.float32),
            scratch_types=[
                pltpu.VMEM((B, t_per_w), jnp.int32),
                pltpu.VMEM((_CHUNK, D), jnp.float32),
                pltpu.VMEM((_CHUNK, D), jnp.float32),
                pltpu.VMEM((_CHUNK, D), jnp.float32),
                pltpu.VMEM((_CHUNK, D), jnp.float32),
                pltpu.VMEM((_CHUNK, D), jnp.float32),
                pltpu.VMEM((_CHUNK, D), jnp.float32),
                pltpu.VMEM_SHARED((NS, t_per_w, D), jnp.float32),
            ] + [pltpu.SemaphoreType.DMA] * 9,
        )
        def k(idx_hbm, table_hbm, pe_hbm, out_hbm,
              idx_v, g0, g1, g2, o0, o1, o2, pe_sh,
              sg0, sg1, sg2, sf0, sf1, sf2, ss0, ss1, ss2):
            cid = lax.axis_index("c")
            sid = lax.axis_index("s")
            wid = sid * NC + cid
            t0 = wid * t_per_w
            pltpu.sync_copy(idx_hbm.at[wid], idx_v)

            gbuf, gsem = (g0, g1, g2), (sg0, sg1, sg2)
            obuf, fsem = (o0, o1, o2), (sf0, sf1, sf2)
            ssem = (ss0, ss1, ss2)

            def loc(c):
                return divmod(c, halves)

            def gather_start(c):
                b, half = loc(c)
                idx = idx_v.at[b, pl.ds(half * _CHUNK, _CHUNK)]
                return pltpu.async_copy(table_hbm.at[idx], gbuf[c % 3],
                                        gsem[c % 3])

            def fill_start(c):
                _, half = loc(c)
                src = pe_sh.at[sid, pl.ds(half * _CHUNK, _CHUNK)]
                return pltpu.async_copy(src, obuf[c % 3], fsem[c % 3])

            def store_start(c):
                b, half = loc(c)
                dst = out_hbm.at[pl.ds(b * T + t0 + half * _CHUNK, _CHUNK)]
                return pltpu.async_copy(obuf[c % 3], dst, ssem[c % 3])

            h_g = [None] * n_chunks
            h_f = [None] * n_chunks
            h_s = [None] * n_chunks
            h_g[0] = gather_start(0)
            h_g[1] = gather_start(1)
            h_g[2] = gather_start(2)
            pltpu.sync_copy(pe_hbm.at[pl.ds(t0, t_per_w)], pe_sh.at[sid])
            h_f[0] = fill_start(0)
            h_f[1] = fill_start(1)

            for c in range(n_chunks):
                h_g[c].wait()
                h_f[c].wait()
                g, o = gbuf[c % 3], obuf[c % 3]

                def body(r, _):
                    for j in range(D // _LANES):
                        sl = pl.ds(j * _LANES, _LANES)
                        plsc.addupdate(o.at[r, sl], g[r, sl] * _SCALE)
                    return _

                lax.fori_loop(0, _CHUNK, body, None)
                h_s[c] = store_start(c)
                if c + 3 < n_chunks:
                    h_g[c + 3] = gather_start(c + 3)
                if c + 2 < n_chunks:
                    if c >= 1:
                        h_s[c - 1].wait()
                    h_f[c + 2] = fill_start(c + 2)
            h_s[n_chunks - 3].wait()
            h_s[n_chunks - 2].wait()
            h_s[n_chunks - 1].wait()

        return k(idx_w, table, pe)

    return call


def kernel(token_ids, embedding_table):
    B, T = token_ids.shape
    V, D = embedding_table.shape
    info = plsc.get_sparse_core_info()
    NW = info.num_cores * info.num_subcores
    t_per_w = T // NW
    idx_w = token_ids.reshape(B, NW, t_per_w).transpose(1, 0, 2)
    pe = jnp.asarray(_pos_encoding(T, D))
    call = _make_sc_call(B, T, V, D)
    out = call(idx_w, embedding_table, pe)
    return out.reshape(B, T, D)


# speedup vs baseline: 1.7002x; 1.0128x over previous
import math

import jax
import jax.numpy as jnp
import numpy as np
from jax import lax
from jax.experimental import pallas as pl
from jax.experimental.pallas import tpu as pltpu
from jax.experimental.pallas import tpu_sc as plsc

D_MODEL = 768
_SCALE = math.sqrt(float(D_MODEL))
_LANES = 16
_CHUNK = 16


def _pos_encoding(seq_len: int, d_model: int) -> np.ndarray:
    pos = np.arange(seq_len, dtype=np.float32)[:, None]
    i = np.arange(d_model, dtype=np.float32)[None, :]
    angle_rates = 1.0 / np.power(10000.0, (2.0 * np.floor(i / 2.0)) / d_model)
    angles = pos * angle_rates
    pe = np.zeros((seq_len, d_model), dtype=np.float32)
    pe[:, 0::2] = np.sin(angles[:, 0::2])
    pe[:, 1::2] = np.cos(angles[:, 1::2])
    return pe


def _make_sc_call(B: int, T: int, V: int, D: int):
    info = plsc.get_sparse_core_info()
    NC, NS = info.num_cores, info.num_subcores
    NW = NC * NS
    assert T % NW == 0
    t_per_w = T // NW
    assert t_per_w % _CHUNK == 0
    halves = t_per_w // _CHUNK
    n_chunks = B * halves

    mesh = plsc.VectorSubcoreMesh(core_axis_name="c", subcore_axis_name="s")

    @jax.jit
    def call(idx_w, table, pe):
        @pl.kernel(
            mesh=mesh,
            out_type=jax.ShapeDtypeStruct((B * T, D), jnp.float32),
            scratch_types=[
                pltpu.VMEM((B, t_per_w), jnp.int32),
                pltpu.VMEM((_CHUNK, D), jnp.float32),
                pltpu.VMEM((_CHUNK, D), jnp.float32),
                pltpu.VMEM((_CHUNK, D), jnp.float32),
                pltpu.VMEM((_CHUNK, D), jnp.float32),
                pltpu.VMEM((_CHUNK, D), jnp.float32),
                pltpu.VMEM((_CHUNK, D), jnp.float32),
                pltpu.VMEM_SHARED((NS, t_per_w, D), jnp.float32),
            ] + [pltpu.SemaphoreType.DMA] * 10,
        )
        def k(idx_hbm, table_hbm, pe_hbm, out_hbm,
              idx_v, g0, g1, g2, o0, o1, o2, pe_sh,
              sg0, sg1, sg2, sf0, sf1, sf2, ss0, ss1, ss2, spe):
            cid = lax.axis_index("c")
            sid = lax.axis_index("s")
            wid = sid * NC + cid
            t0 = wid * t_per_w
            pltpu.sync_copy(idx_hbm.at[wid], idx_v)

            gbuf, gsem = (g0, g1, g2), (sg0, sg1, sg2)
            obuf, fsem = (o0, o1, o2), (sf0, sf1, sf2)
            ssem = (ss0, ss1, ss2)

            def loc(c):
                return divmod(c, halves)

            def gather_start(c):
                b, half = loc(c)
                idx = idx_v.at[b, pl.ds(half * _CHUNK, _CHUNK)]
                return pltpu.async_copy(table_hbm.at[idx], gbuf[c % 3],
                                        gsem[c % 3])

            def fill_start(c):
                _, half = loc(c)
                if c < 2:
                    src = pe_hbm.at[pl.ds(t0 + half * _CHUNK, _CHUNK)]
                else:
                    src = pe_sh.at[sid, pl.ds(half * _CHUNK, _CHUNK)]
                return pltpu.async_copy(src, obuf[c % 3], fsem[c % 3])

            def store_start(c):
                b, half = loc(c)
                dst = out_hbm.at[pl.ds(b * T + t0 + half * _CHUNK, _CHUNK)]
                return pltpu.async_copy(obuf[c % 3], dst, ssem[c % 3])

            h_g = [None] * n_chunks
            h_f = [None] * n_chunks
            h_s = [None] * n_chunks
            h_g[0] = gather_start(0)
            h_g[1] = gather_start(1)
            h_g[2] = gather_start(2)
            h_pe = pltpu.async_copy(
                pe_hbm.at[pl.ds(t0, t_per_w)], pe_sh.at[sid], spe)
            h_f[0] = fill_start(0)
            h_f[1] = fill_start(1)

            for c in range(n_chunks):
                h_g[c].wait()
                h_f[c].wait()
                g, o = gbuf[c % 3], obuf[c % 3]

                def body(r, _):
                    for j in range(D // _LANES):
                        sl = pl.ds(j * _LANES, _LANES)
                        plsc.addupdate(o.at[r, sl], g[r, sl] * _SCALE)
                    return _

                lax.fori_loop(0, _CHUNK, body, None)
                h_s[c] = store_start(c)
                if c + 3 < n_chunks:
                    h_g[c + 3] = gather_start(c + 3)
                if c + 2 < n_chunks:
                    if c == 0:
                        h_pe.wait()
                    if c >= 1:
                        h_s[c - 1].wait()
                    h_f[c + 2] = fill_start(c + 2)
            h_s[n_chunks - 3].wait()
            h_s[n_chunks - 2].wait()
            h_s[n_chunks - 1].wait()

        return k(idx_w, table, pe)

    return call


def kernel(token_ids, embedding_table):
    B, T = token_ids.shape
    V, D = embedding_table.shape
    info = plsc.get_sparse_core_info()
    NW = info.num_cores * info.num_subcores
    t_per_w = T // NW
    idx_w = token_ids.reshape(B, NW, t_per_w).transpose(1, 0, 2)
    pe = jnp.asarray(_pos_encoding(T, D))
    call = _make_sc_call(B, T, V, D)
    out = call(idx_w, embedding_table, pe)
    return out.reshape(B, T, D)
